# baseline (speedup 1.0000x reference)
import functools

import jax
import jax.numpy as jnp
from jax.experimental import pallas as pl

_B, _L, _D = 1, 2048, 768
_MAXW = 8
_WE = 128
_P = 64
_NB = _L // _P
_R = _P * _MAXW
_H = _P + _MAXW
_K = _H + _MAXW


def _span_kernel(h_ref, watt_ref, batt_ref, wtab_ref, w1_ref, w2_ref,
                 bdp_ref, out_ref):
    i = pl.program_id(0)
    base = i * _P
    start = jnp.minimum(base, _L - _H)
    delta = base - start

    hh = h_ref[pl.ds(start, _H), :]

    a = jnp.dot(hh, watt_ref[:, :],
                preferred_element_type=jnp.float32) + batt_ref[0, 0]
    e = jnp.exp(a - jnp.max(a))

    g = jnp.dot(hh, w1_ref[:, :], preferred_element_type=jnp.float32)
    ge = e * g

    wt = jnp.dot(wtab_ref[:, :], w2_ref[:, :],
                 preferred_element_type=jnp.float32) + bdp_ref[:, :]

    g_aug = jnp.concatenate([ge, wt], axis=0)
    e_pad = jnp.concatenate([e, jnp.zeros((_MAXW, 1), jnp.float32)], axis=0)

    r_io = jax.lax.broadcasted_iota(jnp.int32, (_R, _K), 0)
    q_io = jax.lax.broadcasted_iota(jnp.int32, (_R, _K), 1)
    p_loc = r_io >> 3
    wv = r_io & 7
    mcap = (_L - 1) - (base + p_loc)
    m = jnp.minimum(wv, mcap)
    d = q_io - (p_loc + delta)
    band = (d >= 0) & (d <= m)

    den = jnp.dot(band.astype(jnp.float32), e_pad,
                  preferred_element_type=jnp.float32)
    recip = 1.0 / (den + 1e-13)

    onehot = (q_io - _H) == m
    a_mat = jnp.where(band, recip, jnp.where(onehot, 1.0, 0.0))

    res = jnp.dot(a_mat, g_aug, preferred_element_type=jnp.float32)
    out_ref[:, :] = jnp.maximum(res, 0.0)


@jax.jit
def _run(h, W_att, b_att, width_table, W_dp, b_dp):
    h2 = h.reshape(_L, _D)
    w1 = W_dp[:_D]
    w2 = W_dp[_D:]
    out = pl.pallas_call(
        _span_kernel,
        grid=(_NB,),
        in_specs=[
            pl.BlockSpec((_L, _D), lambda i: (0, 0)),
            pl.BlockSpec((_D, 1), lambda i: (0, 0)),
            pl.BlockSpec((1, 1), lambda i: (0, 0)),
            pl.BlockSpec((_MAXW, _WE), lambda i: (0, 0)),
            pl.BlockSpec((_D, _D), lambda i: (0, 0)),
            pl.BlockSpec((_WE, _D), lambda i: (0, 0)),
            pl.BlockSpec((1, _D), lambda i: (0, 0)),
        ],
        out_specs=pl.BlockSpec((_R, _D), lambda i: (i, 0)),
        out_shape=jax.ShapeDtypeStruct((_L * _MAXW, _D), jnp.float32),
    )(h2, W_att, b_att.reshape(1, 1), width_table, w1, w2,
      b_dp.reshape(1, _D))
    return out.reshape(_B, _L, _MAXW, _D)


def kernel(h, span_idx, W_att, b_att, width_table, W_dp, b_dp):
    return _run(h, W_att, b_att, width_table, W_dp, b_dp)


# speedup vs baseline: 1.4105x; 1.4105x over previous
import functools

import jax
import jax.numpy as jnp
from jax.experimental import pallas as pl
from jax.experimental.pallas import tpu as pltpu

_B, _L, _D = 1, 2048, 768
_MAXW = 8
_WE = 128
_P = 128
_NB = _L // _P
_R = _P * _MAXW
_H = _P + _MAXW
_K = _H + _MAXW


def _span_kernel(h_ref, watt_ref, batt_ref, wtab_ref, wdp_ref, bdp_ref,
                 out_ref, band_scr, oh_scr):
    i = pl.program_id(0)
    base = i * _P
    start = jnp.minimum(base, _L - _H)
    delta = base - start

    @pl.when((i == 0) | (i == _NB - 1))
    def _build_masks():
        r_io = jax.lax.broadcasted_iota(jnp.int32, (_R, _K), 0)
        q_io = jax.lax.broadcasted_iota(jnp.int32, (_R, _K), 1)
        p_loc = r_io >> 3
        wv = r_io & 7
        mcap = (_L - 1) - (base + p_loc)
        m = jnp.minimum(wv, mcap)
        d = q_io - (p_loc + delta)
        band = (d >= 0) & (d <= m)
        band_scr[:, :] = band.astype(jnp.float32)
        oh_scr[:, :] = ((q_io - _H) == m).astype(jnp.float32)

    hh = h_ref[pl.ds(start, _H), :]

    a = jnp.dot(hh, watt_ref[:, :],
                preferred_element_type=jnp.float32) + batt_ref[0, 0]
    e = jnp.exp(a - jnp.max(a))

    g = jnp.dot(hh, wdp_ref[0:_D, :], preferred_element_type=jnp.float32)
    ge = e * g

    wt = jnp.dot(wtab_ref[:, :], wdp_ref[_D:_D + _WE, :],
                 preferred_element_type=jnp.float32) + bdp_ref[:, :]

    g_aug = jnp.concatenate([ge, wt], axis=0)
    e_pad = jnp.concatenate([e, jnp.zeros((_MAXW, 1), jnp.float32)], axis=0)

    bandf = band_scr[:, :]
    den = jnp.dot(bandf, e_pad,
                  preferred_element_type=jnp.float32)
    recip = 1.0 / (den + 1e-13)

    a_mat = bandf * recip + oh_scr[:, :]

    res = jnp.dot(a_mat, g_aug, preferred_element_type=jnp.float32)
    out_ref[:, :] = jnp.maximum(res, 0.0)


@jax.jit
def _run(h, W_att, b_att, width_table, W_dp, b_dp):
    h2 = h.reshape(_L, _D)
    out = pl.pallas_call(
        _span_kernel,
        grid=(_NB,),
        in_specs=[
            pl.BlockSpec((_L, _D), lambda i: (0, 0)),
            pl.BlockSpec((_D, 1), lambda i: (0, 0)),
            pl.BlockSpec((1, 1), lambda i: (0, 0)),
            pl.BlockSpec((_MAXW, _WE), lambda i: (0, 0)),
            pl.BlockSpec((_D + _WE, _D), lambda i: (0, 0)),
            pl.BlockSpec((1, _D), lambda i: (0, 0)),
        ],
        out_specs=pl.BlockSpec((_R, _D), lambda i: (i, 0)),
        out_shape=jax.ShapeDtypeStruct((_L * _MAXW, _D), jnp.float32),
        scratch_shapes=[
            pltpu.VMEM((_R, _K), jnp.float32),
            pltpu.VMEM((_R, _K), jnp.float32),
        ],
    )(h2, W_att, b_att.reshape(1, 1), width_table, W_dp,
      b_dp.reshape(1, _D))
    return out.reshape(_B, _L, _MAXW, _D)


def kernel(h, span_idx, W_att, b_att, width_table, W_dp, b_dp):
    return _run(h, W_att, b_att, width_table, W_dp, b_dp)
